# Initial kernel scaffold; baseline (speedup 1.0000x reference)
import jax
import jax.numpy as jnp
from jax.experimental import pallas as pl


def kernel(input_ids, word_emb, pos_emb, type_emb, ln_gamma, ln_beta):
    raise NotImplementedError("write your pallas kernel here")


# speedup vs baseline: 1.2090x; 1.2090x over previous
import functools

import jax
import jax.numpy as jnp
from jax import lax
from jax.experimental import pallas as pl
from jax.experimental.pallas import tpu as pltpu
from jax.experimental.pallas import tpu_sc as plsc

B, L, D = 1024, 50, 768
N = B * L
EPS = 1e-12

NC, NS = 2, 16
NW = NC * NS
TPW = N // NW
CHUNK = 80
NCHUNK = TPW // CHUNK


def _sc_gather(ids3, table):
    mesh = plsc.VectorSubcoreMesh(core_axis_name="c", subcore_axis_name="s")

    @functools.partial(
        pl.kernel,
        mesh=mesh,
        out_type=jax.ShapeDtypeStruct((N, D), jnp.float32),
        scratch_types=[
            pltpu.VMEM((NCHUNK, CHUNK), jnp.int32),
            pltpu.VMEM((CHUNK, D), jnp.float32),
            pltpu.VMEM((CHUNK, D), jnp.float32),
            pltpu.SemaphoreType.DMA,
            pltpu.SemaphoreType.DMA,
            pltpu.SemaphoreType.DMA,
            pltpu.SemaphoreType.DMA,
        ],
    )
    def k(ids_hbm, table_hbm, out_hbm, idx_v, rows_a, rows_b, gs_a, gs_b,
          os_a, os_b):
        wid = lax.axis_index("s") * NC + lax.axis_index("c")
        base = wid * TPW

        pltpu.sync_copy(ids_hbm.at[wid], idx_v)

        rows = (rows_a, rows_b)
        gsem = (gs_a, gs_b)
        osem = (os_a, os_b)

        def gather_start(k_chunk, buf):
            pltpu.make_async_copy(
                table_hbm.at[idx_v.at[k_chunk]], rows[buf], gsem[buf]
            ).start()

        def gather_wait(buf):
            pltpu.make_async_copy(
                table_hbm.at[idx_v.at[0]], rows[buf], gsem[buf]
            ).wait()

        def out_start(k_chunk, buf):
            pltpu.make_async_copy(
                rows[buf], out_hbm.at[pl.ds(base + k_chunk * CHUNK, CHUNK)],
                osem[buf],
            ).start()

        def out_wait(buf):
            pltpu.make_async_copy(
                rows[buf], out_hbm.at[pl.ds(base, CHUNK)], osem[buf]
            ).wait()

        gather_start(0, 0)
        gather_start(1, 1)

        def body(kk, carry):
            for b in range(2):
                k_chunk = kk * 2 + b
                gather_wait(b)
                out_start(k_chunk, b)
                out_wait(b)
                gather_start(k_chunk + 2, b)
            return carry

        lax.fori_loop(0, NCHUNK // 2 - 1, body, 0)

        for b in range(2):
            k_chunk = NCHUNK - 2 + b
            gather_wait(b)
            out_start(k_chunk, b)
            out_wait(b)

    return k(ids3, table)


SEQ_BLK = 16


def _ln_body(x_ref, padd_ref, g_ref, bta_ref, o_ref):
    e = x_ref[...] + padd_ref[...][None, :, :]
    mu = jnp.mean(e, axis=-1, keepdims=True)
    d = e - mu
    var = jnp.mean(d * d, axis=-1, keepdims=True)
    o_ref[...] = d * lax.rsqrt(var + EPS) * g_ref[...][None, :, :] \
        + bta_ref[...][None, :, :]


def _tc_add_ln(gathered3, padd, gamma2, beta2):
    return pl.pallas_call(
        _ln_body,
        grid=(B // SEQ_BLK,),
        in_specs=[
            pl.BlockSpec((SEQ_BLK, L, D), lambda i: (i, 0, 0)),
            pl.BlockSpec((L, D), lambda i: (0, 0)),
            pl.BlockSpec((1, D), lambda i: (0, 0)),
            pl.BlockSpec((1, D), lambda i: (0, 0)),
        ],
        out_specs=pl.BlockSpec((SEQ_BLK, L, D), lambda i: (i, 0, 0)),
        out_shape=jax.ShapeDtypeStruct((B, L, D), jnp.float32),
        compiler_params=pltpu.CompilerParams(
            dimension_semantics=("arbitrary",),
        ),
    )(gathered3, padd, gamma2, beta2)


def kernel(input_ids, word_emb, pos_emb, type_emb, ln_gamma, ln_beta):
    ids3 = input_ids.astype(jnp.int32).reshape(NW, NCHUNK, CHUNK)
    gathered = _sc_gather(ids3, word_emb)
    padd = pos_emb[:L] + type_emb[0][None, :]
    out = _tc_add_ln(
        gathered.reshape(B, L, D),
        padd,
        ln_gamma.reshape(1, D),
        ln_beta.reshape(1, D),
    )
    return out
